# baseline (speedup 1.0000x reference)
import functools

import jax
import jax.numpy as jnp
from jax import lax
from jax.experimental import pallas as pl
from jax.experimental.pallas import tpu as pltpu
from jax.experimental.pallas import tpu_sc as plsc

BATCH = 16384
EMBED_DIM = 64

_info = plsc.get_sparse_core_info()
_NW = _info.num_cores * _info.num_subcores
_R_PER_W = BATCH // _NW
_LANES = 16
_CHUNK = 128
_N_CHUNKS = _R_PER_W // _CHUNK


@functools.partial(
    pl.kernel,
    mesh=plsc.VectorSubcoreMesh(core_axis_name="c", subcore_axis_name="s"),
    out_type=jax.ShapeDtypeStruct((BATCH, EMBED_DIM), jnp.float32),
    scratch_types=[
        pltpu.VMEM((2, EMBED_DIM), jnp.float32),
        pltpu.VMEM((_R_PER_W,), jnp.int32),
        pltpu.VMEM((_R_PER_W, EMBED_DIM), jnp.float32),
        pltpu.SemaphoreType.DMA,
    ],
    compiler_params=pltpu.CompilerParams(needs_layout_passes=False),
)
def _role_lookup(w_hbm, role_hbm, out_hbm, w_v, role_v, rows_v, sem):
    wid = lax.axis_index("s") * _info.num_cores + lax.axis_index("c")
    base = wid * _R_PER_W
    pltpu.sync_copy(w_hbm, w_v)
    pltpu.sync_copy(role_hbm.at[pl.ds(base, _R_PER_W)], role_v)

    lane = lax.iota(jnp.int32, _LANES)

    def group_body(k, _):
        r0 = _LANES * k
        roles = role_v[pl.ds(r0, _LANES)]
        rows16 = r0 + lane
        for c in range(EMBED_DIM):
            cvec = jnp.full((_LANES,), c, jnp.int32)
            v = plsc.load_gather(w_v, [roles, cvec])
            plsc.store_scatter(rows_v, [rows16, cvec], v)
        return 0

    writes = []
    for j in range(_N_CHUNKS):
        lax.fori_loop(
            j * _CHUNK // _LANES, (j + 1) * _CHUNK // _LANES, group_body, 0
        )
        writes.append(
            pltpu.async_copy(
                rows_v.at[pl.ds(j * _CHUNK, _CHUNK)],
                out_hbm.at[pl.ds(base + j * _CHUNK, _CHUNK)],
                sem,
            )
        )
    for w in writes:
        w.wait()


def kernel(obs, role, W_role):
    del obs
    return _role_lookup(W_role, role)
